# baseline (speedup 1.0000x reference)
import functools

import jax
import jax.numpy as jnp
from jax import lax
from jax.experimental import pallas as pl
from jax.experimental.pallas import tpu as pltpu
from jax.experimental.pallas import tpu_sc as plsc

B = 16384
L_PN = 20
L_TNC = 200
V_PN = 1000
V_TNC = 10000
E = 8
EP = E // 2
HP = L_PN // 2
HT = L_TNC // 2
D_OUT = 32

NC = 2
NS = 16
LANES = 16
NW = NC * NS
ROWS_PER_W = B // NW
SG_ROWS = 64
NSG = ROWS_PER_W // SG_ROWS
NGRP = SG_ROWS // LANES


def _unpack2(g):
  lo = plsc.bitcast(lax.shift_left(g, 16), jnp.float32)
  hi = plsc.bitcast(lax.bitwise_and(g, -65536), jnp.float32)
  return lo, hi


def _make_kernel():
  mesh = plsc.VectorSubcoreMesh(core_axis_name="c", subcore_axis_name="s",
                                num_cores=NC)

  @functools.partial(
      pl.kernel,
      out_type=jax.ShapeDtypeStruct((B * D_OUT,), jnp.float32),
      mesh=mesh,
      compiler_params=pltpu.CompilerParams(
          needs_layout_passes=False, use_tc_tiling_on_sc=False),
      scratch_types=[
          pltpu.VMEM((V_PN * EP,), jnp.int32),
          pltpu.VMEM((V_TNC * EP,), jnp.int32),
          pltpu.VMEM((2, SG_ROWS * HP), jnp.int32),
          pltpu.VMEM((2, SG_ROWS * HT), jnp.int32),
          pltpu.VMEM((E * D_OUT * LANES,), jnp.int32),
          pltpu.VMEM((D_OUT * LANES,), jnp.float32),
          pltpu.VMEM((2, SG_ROWS * D_OUT), jnp.float32),
          pltpu.SemaphoreType.DMA,
          pltpu.SemaphoreType.DMA,
          pltpu.SemaphoreType.DMA,
          pltpu.SemaphoreType.DMA,
          pltpu.SemaphoreType.DMA,
          pltpu.SemaphoreType.DMA,
      ],
  )
  def k(pn_tok_hbm, tnc_tok_hbm, pt_hbm, tt_hbm, wb_hbm, bb_hbm, out_hbm,
        pt_v, tt_v, pn_v, tnc_v, wb_v, bb_v, out_v,
        sem_pn0, sem_pn1, sem_tnc0, sem_tnc1, sem_out0, sem_out1):
    sem_pn = (sem_pn0, sem_pn1)
    sem_tnc = (sem_tnc0, sem_tnc1)
    sem_out = (sem_out0, sem_out1)

    wid = lax.axis_index("s") * NC + lax.axis_index("c")
    rbase0 = wid * ROWS_PER_W

    pltpu.sync_copy(pt_hbm, pt_v)
    pltpu.sync_copy(tt_hbm, tt_v)
    pltpu.sync_copy(wb_hbm, wb_v)
    pltpu.sync_copy(bb_hbm, bb_v)

    row_iota = lax.iota(jnp.int32, LANES)
    zero = jnp.zeros((LANES,), jnp.float32)

    def pn_slice(s):
      return pn_tok_hbm.at[pl.ds((rbase0 + s * SG_ROWS) * HP, SG_ROWS * HP)]

    def tnc_slice(s):
      return tnc_tok_hbm.at[pl.ds((rbase0 + s * SG_ROWS) * HT, SG_ROWS * HT)]

    def out_slice(s):
      return out_hbm.at[pl.ds((rbase0 + s * SG_ROWS) * D_OUT,
                              SG_ROWS * D_OUT)]

    def pool(tok_ref, table_ref, half_l, row_h):
      def body(h, accs):
        w = plsc.load_gather(tok_ref, [row_h + h])
        tok_a = lax.bitwise_and(w, 65535)
        tok_b = lax.shift_right_logical(w, 16)
        ta = tok_a * EP
        tb = tok_b * EP
        accs = list(accs)
        for jj in range(EP):
          ga = plsc.load_gather(table_ref, [ta + jj])
          lo, hi = _unpack2(ga)
          accs[2 * jj] = accs[2 * jj] + lo
          accs[2 * jj + 1] = accs[2 * jj + 1] + hi
          gb = plsc.load_gather(table_ref, [tb + jj])
          lo, hi = _unpack2(gb)
          accs[E + 2 * jj] = accs[E + 2 * jj] + lo
          accs[E + 2 * jj + 1] = accs[E + 2 * jj + 1] + hi
        return tuple(accs)
      accs = plsc.parallel_loop(0, half_l, 1, unroll=4,
                                carry=(zero,) * (2 * E))(body)
      return tuple(accs[j] + accs[E + j] for j in range(E))

    for b in range(2):
      pltpu.async_copy(pn_slice(b), pn_v.at[b], sem_pn[b])
      pltpu.async_copy(tnc_slice(b), tnc_v.at[b], sem_tnc[b])

    def outer(o, _):
      for b in range(2):
        s = 2 * o + b
        pltpu.make_async_copy(pn_slice(s), pn_v.at[b], sem_pn[b]).wait()
        pltpu.make_async_copy(tnc_slice(s), tnc_v.at[b], sem_tnc[b]).wait()

        @pl.when(o >= 1)
        def _wait_out():
          pltpu.make_async_copy(out_v.at[b], out_slice(s), sem_out[b]).wait()

        def group_body(g, _):
          rb = g * LANES
          row_pn = row_iota * HP + rb * HP
          row_tnc = row_iota * HT + rb * HT
          acc_pn = pool(pn_v.at[b], pt_v, HP, row_pn)
          acc_tnc = pool(tnc_v.at[b], tt_v, HT, row_tnc)
          feats = acc_pn + acc_tnc

          orow = row_iota * D_OUT + rb * D_OUT

          def dense_body(j, _):
            parts = [bb_v[pl.ds(j * LANES, LANES)], None, None, None]
            for m in range(E):
              wlo, whi = _unpack2(
                  wb_v[pl.ds((m * D_OUT + j) * LANES, LANES)])
              t = feats[2 * m] * wlo + feats[2 * m + 1] * whi
              parts[m % 4] = t if parts[m % 4] is None else parts[m % 4] + t
            oj = (parts[0] + parts[1]) + (parts[2] + parts[3])
            oj = jnp.maximum(oj, 0.0)
            plsc.store_scatter(out_v.at[b], [orow + j], oj)
            return 0

          lax.fori_loop(0, D_OUT, dense_body, 0, unroll=4)
          return 0

        lax.fori_loop(0, NGRP, group_body, 0)

        pltpu.async_copy(out_v.at[b], out_slice(s), sem_out[b])

        @pl.when(s + 2 < NSG)
        def _prefetch():
          pltpu.async_copy(pn_slice(s + 2), pn_v.at[b], sem_pn[b])
          pltpu.async_copy(tnc_slice(s + 2), tnc_v.at[b], sem_tnc[b])
      return 0

    lax.fori_loop(0, NSG // 2, outer, 0)

    for b in range(2):
      pltpu.make_async_copy(out_v.at[b], out_slice(b), sem_out[b]).wait()

  return k


_sc_kernel = _make_kernel()


def _pack_bf16_pairs(x):
  xb = x.astype(jnp.bfloat16).reshape(x.shape[0], -1, 2)
  return lax.bitcast_convert_type(xb, jnp.int32).reshape(-1)


def _pack_tok_pairs(t):
  t16 = t.astype(jnp.int16).reshape(t.shape[0], -1, 2)
  return lax.bitcast_convert_type(t16, jnp.int32).reshape(-1)


@jax.jit
def kernel(program_name_tokens, tnc_tokens, program_table, tnc_table, W, b):
  scale = jnp.concatenate([
      jnp.full((E,), 1.0 / L_PN, jnp.float32),
      jnp.full((E,), 1.0 / L_TNC, jnp.float32),
  ])
  Ws = W * scale[:, None]
  wpairs = Ws.reshape(E, 2, D_OUT).transpose(0, 2, 1)
  wp = _pack_bf16_pairs(wpairs.reshape(E * D_OUT, 2))
  wb = jnp.tile(wp[:, None], (1, LANES)).reshape(-1)
  bb = jnp.tile(b[:, None], (1, LANES)).reshape(-1)
  out = _sc_kernel(
      _pack_tok_pairs(program_name_tokens),
      _pack_tok_pairs(tnc_tokens),
      _pack_bf16_pairs(program_table),
      _pack_bf16_pairs(tnc_table),
      wb,
      bb,
  )
  return out.reshape(B, D_OUT)


# speedup vs baseline: 1.0071x; 1.0071x over previous
import functools

import jax
import jax.numpy as jnp
from jax import lax
from jax.experimental import pallas as pl
from jax.experimental.pallas import tpu as pltpu
from jax.experimental.pallas import tpu_sc as plsc

B = 16384
L_PN = 20
L_TNC = 200
V_PN = 1000
V_TNC = 10000
E = 8
EP = E // 2
HP = L_PN // 2
HT = L_TNC // 2
D_OUT = 32

NC = 2
NS = 16
LANES = 16
NW = NC * NS
ROWS_PER_W = B // NW
SG_ROWS = 64
NSG = ROWS_PER_W // SG_ROWS
NGRP = SG_ROWS // LANES


def _unpack2(g):
  lo = plsc.bitcast(lax.shift_left(g, 16), jnp.float32)
  hi = plsc.bitcast(lax.bitwise_and(g, -65536), jnp.float32)
  return lo, hi


def _make_kernel():
  mesh = plsc.VectorSubcoreMesh(core_axis_name="c", subcore_axis_name="s",
                                num_cores=NC)

  @functools.partial(
      pl.kernel,
      out_type=jax.ShapeDtypeStruct((B * D_OUT,), jnp.float32),
      mesh=mesh,
      compiler_params=pltpu.CompilerParams(
          needs_layout_passes=False, use_tc_tiling_on_sc=False),
      scratch_types=[
          pltpu.VMEM((V_PN * EP,), jnp.int32),
          pltpu.VMEM((V_TNC * EP,), jnp.int32),
          pltpu.VMEM((2, SG_ROWS * HP), jnp.int32),
          pltpu.VMEM((2, SG_ROWS * HT), jnp.int32),
          pltpu.VMEM((E * D_OUT * LANES,), jnp.int32),
          pltpu.VMEM((D_OUT * LANES,), jnp.float32),
          pltpu.VMEM((2, SG_ROWS * D_OUT), jnp.float32),
          pltpu.SemaphoreType.DMA,
          pltpu.SemaphoreType.DMA,
          pltpu.SemaphoreType.DMA,
          pltpu.SemaphoreType.DMA,
          pltpu.SemaphoreType.DMA,
          pltpu.SemaphoreType.DMA,
      ],
  )
  def k(pn_tok_hbm, tnc_tok_hbm, pt_hbm, tt_hbm, wb_hbm, bb_hbm, out_hbm,
        pt_v, tt_v, pn_v, tnc_v, wb_v, bb_v, out_v,
        sem_pn0, sem_pn1, sem_tnc0, sem_tnc1, sem_out0, sem_out1):
    sem_pn = (sem_pn0, sem_pn1)
    sem_tnc = (sem_tnc0, sem_tnc1)
    sem_out = (sem_out0, sem_out1)

    wid = lax.axis_index("s") * NC + lax.axis_index("c")
    rbase0 = wid * ROWS_PER_W

    pltpu.sync_copy(pt_hbm, pt_v)
    pltpu.sync_copy(tt_hbm, tt_v)
    pltpu.sync_copy(wb_hbm, wb_v)
    pltpu.sync_copy(bb_hbm, bb_v)

    row_iota = lax.iota(jnp.int32, LANES)
    zero = jnp.zeros((LANES,), jnp.float32)

    def pn_slice(s):
      return pn_tok_hbm.at[pl.ds((rbase0 + s * SG_ROWS) * HP, SG_ROWS * HP)]

    def tnc_slice(s):
      return tnc_tok_hbm.at[pl.ds((rbase0 + s * SG_ROWS) * HT, SG_ROWS * HT)]

    def out_slice(s):
      return out_hbm.at[pl.ds((rbase0 + s * SG_ROWS) * D_OUT,
                              SG_ROWS * D_OUT)]

    def pool(tok_ref, table_ref, half_l, row_h):
      def body(h, accs):
        w = plsc.load_gather(tok_ref, [row_h + h])
        tok_a = lax.bitwise_and(w, 65535)
        tok_b = lax.shift_right_logical(w, 16)
        ta = tok_a * EP
        tb = tok_b * EP
        accs = list(accs)
        for jj in range(EP):
          ga = plsc.load_gather(table_ref, [ta + jj])
          gb = plsc.load_gather(table_ref, [tb + jj])
          lo_a, hi_a = _unpack2(ga)
          lo_b, hi_b = _unpack2(gb)
          accs[2 * jj] = accs[2 * jj] + (lo_a + lo_b)
          accs[2 * jj + 1] = accs[2 * jj + 1] + (hi_a + hi_b)
        return tuple(accs)
      return plsc.parallel_loop(0, half_l, 1, unroll=4,
                                carry=(zero,) * E)(body)

    for b in range(2):
      pltpu.async_copy(pn_slice(b), pn_v.at[b], sem_pn[b])
      pltpu.async_copy(tnc_slice(b), tnc_v.at[b], sem_tnc[b])

    def outer(o, _):
      for b in range(2):
        s = 2 * o + b
        pltpu.make_async_copy(pn_slice(s), pn_v.at[b], sem_pn[b]).wait()
        pltpu.make_async_copy(tnc_slice(s), tnc_v.at[b], sem_tnc[b]).wait()

        @pl.when(o >= 1)
        def _wait_out():
          pltpu.make_async_copy(out_v.at[b], out_slice(s), sem_out[b]).wait()

        def group_body(g, _):
          rb = g * LANES
          row_pn = row_iota * HP + rb * HP
          row_tnc = row_iota * HT + rb * HT
          acc_pn = pool(pn_v.at[b], pt_v, HP, row_pn)
          acc_tnc = pool(tnc_v.at[b], tt_v, HT, row_tnc)
          feats = acc_pn + acc_tnc

          orow = row_iota * D_OUT + rb * D_OUT

          def dense_body(j, _):
            parts = [bb_v[pl.ds(j * LANES, LANES)], None, None, None]
            for m in range(E):
              wlo, whi = _unpack2(
                  wb_v[pl.ds((m * D_OUT + j) * LANES, LANES)])
              t = feats[2 * m] * wlo + feats[2 * m + 1] * whi
              parts[m % 4] = t if parts[m % 4] is None else parts[m % 4] + t
            oj = (parts[0] + parts[1]) + (parts[2] + parts[3])
            oj = jnp.maximum(oj, 0.0)
            plsc.store_scatter(out_v.at[b], [orow + j], oj)
            return 0

          lax.fori_loop(0, D_OUT, dense_body, 0, unroll=4)
          return 0

        lax.fori_loop(0, NGRP, group_body, 0)

        pltpu.async_copy(out_v.at[b], out_slice(s), sem_out[b])

        @pl.when(s + 2 < NSG)
        def _prefetch():
          pltpu.async_copy(pn_slice(s + 2), pn_v.at[b], sem_pn[b])
          pltpu.async_copy(tnc_slice(s + 2), tnc_v.at[b], sem_tnc[b])
      return 0

    lax.fori_loop(0, NSG // 2, outer, 0)

    for b in range(2):
      pltpu.make_async_copy(out_v.at[b], out_slice(b), sem_out[b]).wait()

  return k


_sc_kernel = _make_kernel()


def _pack_bf16_pairs(x):
  xb = x.astype(jnp.bfloat16).reshape(x.shape[0], -1, 2)
  return lax.bitcast_convert_type(xb, jnp.int32).reshape(-1)


def _pack_tok_pairs(t):
  t16 = t.astype(jnp.int16).reshape(t.shape[0], -1, 2)
  return lax.bitcast_convert_type(t16, jnp.int32).reshape(-1)


@jax.jit
def kernel(program_name_tokens, tnc_tokens, program_table, tnc_table, W, b):
  scale = jnp.concatenate([
      jnp.full((E,), 1.0 / L_PN, jnp.float32),
      jnp.full((E,), 1.0 / L_TNC, jnp.float32),
  ])
  Ws = W * scale[:, None]
  wpairs = Ws.reshape(E, 2, D_OUT).transpose(0, 2, 1)
  wp = _pack_bf16_pairs(wpairs.reshape(E * D_OUT, 2))
  wb = jnp.tile(wp[:, None], (1, LANES)).reshape(-1)
  bb = jnp.tile(b[:, None], (1, LANES)).reshape(-1)
  out = _sc_kernel(
      _pack_tok_pairs(program_name_tokens),
      _pack_tok_pairs(tnc_tokens),
      _pack_bf16_pairs(program_table),
      _pack_bf16_pairs(tnc_table),
      wb,
      bb,
  )
  return out.reshape(B, D_OUT)


# speedup vs baseline: 1.0632x; 1.0557x over previous
import functools

import jax
import jax.numpy as jnp
from jax import lax
from jax.experimental import pallas as pl
from jax.experimental.pallas import tpu as pltpu
from jax.experimental.pallas import tpu_sc as plsc

B = 16384
L_PN = 20
L_TNC = 200
V_PN = 1000
V_TNC = 10000
E = 8
EP = E // 2
HP = L_PN // 2
HT = L_TNC // 2
D_OUT = 32

NC = 2
NS = 16
LANES = 16
NW = NC * NS
ROWS_PER_W = B // NW
SG_ROWS = 64
NSG = ROWS_PER_W // SG_ROWS
NGRP = SG_ROWS // LANES


def _unpack2(g):
  lo = plsc.bitcast(lax.shift_left(g, 16), jnp.float32)
  hi = plsc.bitcast(lax.bitwise_and(g, -65536), jnp.float32)
  return lo, hi


def _make_kernel():
  mesh = plsc.VectorSubcoreMesh(core_axis_name="c", subcore_axis_name="s",
                                num_cores=NC)

  @functools.partial(
      pl.kernel,
      out_type=jax.ShapeDtypeStruct((B * D_OUT,), jnp.float32),
      mesh=mesh,
      compiler_params=pltpu.CompilerParams(
          needs_layout_passes=False, use_tc_tiling_on_sc=False),
      scratch_types=[
          pltpu.VMEM((V_PN * EP,), jnp.int32),
          pltpu.VMEM((V_TNC * EP,), jnp.int32),
          pltpu.VMEM((2, SG_ROWS * HP), jnp.int32),
          pltpu.VMEM((2, SG_ROWS * HT), jnp.int32),
          pltpu.VMEM((E * D_OUT * LANES,), jnp.int32),
          pltpu.VMEM((D_OUT * LANES,), jnp.float32),
          pltpu.VMEM((2, SG_ROWS * D_OUT), jnp.float32),
          pltpu.SemaphoreType.DMA,
          pltpu.SemaphoreType.DMA,
          pltpu.SemaphoreType.DMA,
          pltpu.SemaphoreType.DMA,
          pltpu.SemaphoreType.DMA,
          pltpu.SemaphoreType.DMA,
      ],
  )
  def k(pn_tok_hbm, tnc_tok_hbm, pt_hbm, tt_hbm, wb_hbm, bb_hbm, out_hbm,
        pt_v, tt_v, pn_v, tnc_v, wb_v, bb_v, out_v,
        sem_pn0, sem_pn1, sem_tnc0, sem_tnc1, sem_out0, sem_out1):
    sem_pn = (sem_pn0, sem_pn1)
    sem_tnc = (sem_tnc0, sem_tnc1)
    sem_out = (sem_out0, sem_out1)

    wid = lax.axis_index("s") * NC + lax.axis_index("c")
    rbase0 = wid * ROWS_PER_W

    pltpu.sync_copy(pt_hbm, pt_v)
    pltpu.sync_copy(tt_hbm, tt_v)
    pltpu.sync_copy(wb_hbm, wb_v)
    pltpu.sync_copy(bb_hbm, bb_v)

    row_iota = lax.iota(jnp.int32, LANES)
    zero = jnp.zeros((LANES,), jnp.float32)

    def pn_slice(s):
      return pn_tok_hbm.at[pl.ds((rbase0 + s * SG_ROWS) * HP, SG_ROWS * HP)]

    def tnc_slice(s):
      return tnc_tok_hbm.at[pl.ds((rbase0 + s * SG_ROWS) * HT, SG_ROWS * HT)]

    def out_slice(s):
      return out_hbm.at[pl.ds((rbase0 + s * SG_ROWS) * D_OUT,
                              SG_ROWS * D_OUT)]

    def pool(tok_ref, table_ref, half_l, row_h):
      def body(h, accs):
        w = plsc.load_gather(tok_ref, [row_h + h])
        tok_a = lax.bitwise_and(w, 65535)
        tok_b = lax.shift_right_logical(w, 16)
        ta = tok_a * EP
        tb = tok_b * EP
        accs = list(accs)
        for jj in range(EP):
          ga = plsc.load_gather(table_ref, [ta + jj])
          gb = plsc.load_gather(table_ref, [tb + jj])
          lo_a, hi_a = _unpack2(ga)
          lo_b, hi_b = _unpack2(gb)
          accs[2 * jj] = accs[2 * jj] + (lo_a + lo_b)
          accs[2 * jj + 1] = accs[2 * jj + 1] + (hi_a + hi_b)
        return tuple(accs)
      return plsc.parallel_loop(0, half_l, 1, unroll=4,
                                carry=(zero,) * E)(body)

    for b in range(2):
      pltpu.async_copy(pn_slice(b), pn_v.at[b], sem_pn[b])
      pltpu.async_copy(tnc_slice(b), tnc_v.at[b], sem_tnc[b])

    def outer(o, _):
      for b in range(2):
        s = 2 * o + b
        pltpu.make_async_copy(pn_slice(s), pn_v.at[b], sem_pn[b]).wait()
        pltpu.make_async_copy(tnc_slice(s), tnc_v.at[b], sem_tnc[b]).wait()

        @pl.when(o >= 1)
        def _wait_out():
          pltpu.make_async_copy(out_v.at[b], out_slice(s), sem_out[b]).wait()

        def group_body(g, _):
          rb = g * LANES
          row_pn = row_iota * HP + rb * HP
          row_tnc = row_iota * HT + rb * HT
          acc_pn = pool(pn_v.at[b], pt_v, HP, row_pn)
          acc_tnc = pool(tnc_v.at[b], tt_v, HT, row_tnc)
          feats = acc_pn + acc_tnc

          orow = row_iota * D_OUT + rb * D_OUT

          def dense_body(j, _):
            parts = [bb_v[pl.ds(j * LANES, LANES)], None, None, None]
            for m in range(E):
              wlo, whi = _unpack2(
                  wb_v[pl.ds((m * D_OUT + j) * LANES, LANES)])
              t = feats[2 * m] * wlo + feats[2 * m + 1] * whi
              parts[m % 4] = t if parts[m % 4] is None else parts[m % 4] + t
            oj = (parts[0] + parts[1]) + (parts[2] + parts[3])
            oj = jnp.maximum(oj, 0.0)
            plsc.store_scatter(out_v.at[b], [orow + j], oj)
            return 0

          lax.fori_loop(0, D_OUT, dense_body, 0, unroll=4)
          return 0

        lax.fori_loop(0, NGRP, group_body, 0)

        pltpu.async_copy(out_v.at[b], out_slice(s), sem_out[b])

        @pl.when(s + 2 < NSG)
        def _prefetch():
          pltpu.async_copy(pn_slice(s + 2), pn_v.at[b], sem_pn[b])
          pltpu.async_copy(tnc_slice(s + 2), tnc_v.at[b], sem_tnc[b])
      return 0

    lax.fori_loop(0, NSG // 2, outer, 0)

    for b in range(2):
      pltpu.make_async_copy(out_v.at[b], out_slice(b), sem_out[b]).wait()

  return k


_sc_kernel = _make_kernel()


def _pack_bf16_pairs(x):
  xb = x.astype(jnp.bfloat16).reshape(x.shape[0], -1, 2)
  return lax.bitcast_convert_type(xb, jnp.int32).reshape(-1)


def _pack_tok_pairs(t):
  a = t.astype(jnp.int32).reshape(t.shape[0], -1, 2)
  return (a[..., 0] | (a[..., 1] << 16)).reshape(-1)


@jax.jit
def kernel(program_name_tokens, tnc_tokens, program_table, tnc_table, W, b):
  scale = jnp.concatenate([
      jnp.full((E,), 1.0 / L_PN, jnp.float32),
      jnp.full((E,), 1.0 / L_TNC, jnp.float32),
  ])
  Ws = W * scale[:, None]
  wpairs = Ws.reshape(E, 2, D_OUT).transpose(0, 2, 1)
  wp = _pack_bf16_pairs(wpairs.reshape(E * D_OUT, 2))
  wb = jnp.tile(wp[:, None], (1, LANES)).reshape(-1)
  bb = jnp.tile(b[:, None], (1, LANES)).reshape(-1)
  out = _sc_kernel(
      _pack_tok_pairs(program_name_tokens),
      _pack_tok_pairs(tnc_tokens),
      _pack_bf16_pairs(program_table),
      _pack_bf16_pairs(tnc_table),
      wb,
      bb,
  )
  return out.reshape(B, D_OUT)


# speedup vs baseline: 1.4936x; 1.4048x over previous
import functools

import jax
import jax.numpy as jnp
from jax import lax
from jax.experimental import pallas as pl
from jax.experimental.pallas import tpu as pltpu
from jax.experimental.pallas import tpu_sc as plsc

B = 16384
L_PN = 20
L_TNC = 200
V_PN = 1000
V_TNC = 10000
E = 8
EP = E // 2
D_OUT = 32

NC = 2
NS = 16
LANES = 16
NW = NC * NS
ROWS_PER_W = B // NW
SG_ROWS = 64
NSG = ROWS_PER_W // SG_ROWS
NGRP = SG_ROWS // LANES


def _unpack2(g):
  lo = plsc.bitcast(lax.shift_left(g, 16), jnp.float32)
  hi = plsc.bitcast(lax.bitwise_and(g, -65536), jnp.float32)
  return lo, hi


def _make_kernel():
  mesh = plsc.VectorSubcoreMesh(core_axis_name="c", subcore_axis_name="s",
                                num_cores=NC)

  @functools.partial(
      pl.kernel,
      out_type=jax.ShapeDtypeStruct((B * D_OUT,), jnp.float32),
      mesh=mesh,
      compiler_params=pltpu.CompilerParams(
          needs_layout_passes=False, use_tc_tiling_on_sc=False),
      scratch_types=[
          pltpu.VMEM((V_PN * EP,), jnp.int32),
          pltpu.VMEM((V_TNC * EP,), jnp.int32),
          pltpu.VMEM((2, SG_ROWS * L_PN), jnp.int32),
          pltpu.VMEM((2, SG_ROWS * L_TNC), jnp.int32),
          pltpu.VMEM((E * D_OUT * LANES,), jnp.int32),
          pltpu.VMEM((D_OUT * LANES,), jnp.float32),
          pltpu.VMEM((2, SG_ROWS * D_OUT), jnp.float32),
          pltpu.SemaphoreType.DMA,
          pltpu.SemaphoreType.DMA,
          pltpu.SemaphoreType.DMA,
          pltpu.SemaphoreType.DMA,
          pltpu.SemaphoreType.DMA,
          pltpu.SemaphoreType.DMA,
      ],
  )
  def k(pn_tok_hbm, tnc_tok_hbm, pt_hbm, tt_hbm, wb_hbm, bb_hbm, out_hbm,
        pt_v, tt_v, pn_v, tnc_v, wb_v, bb_v, out_v,
        sem_pn0, sem_pn1, sem_tnc0, sem_tnc1, sem_out0, sem_out1):
    sem_pn = (sem_pn0, sem_pn1)
    sem_tnc = (sem_tnc0, sem_tnc1)
    sem_out = (sem_out0, sem_out1)

    wid = lax.axis_index("s") * NC + lax.axis_index("c")
    rbase0 = wid * ROWS_PER_W

    pltpu.sync_copy(pt_hbm, pt_v)
    pltpu.sync_copy(tt_hbm, tt_v)
    pltpu.sync_copy(wb_hbm, wb_v)
    pltpu.sync_copy(bb_hbm, bb_v)

    row_iota = lax.iota(jnp.int32, LANES)
    zero = jnp.zeros((LANES,), jnp.float32)

    def pn_slice(s):
      return pn_tok_hbm.at[pl.ds((rbase0 + s * SG_ROWS) * L_PN,
                                 SG_ROWS * L_PN)]

    def tnc_slice(s):
      return tnc_tok_hbm.at[pl.ds((rbase0 + s * SG_ROWS) * L_TNC,
                                  SG_ROWS * L_TNC)]

    def out_slice(s):
      return out_hbm.at[pl.ds((rbase0 + s * SG_ROWS) * D_OUT,
                              SG_ROWS * D_OUT)]

    def pool(tok_ref, table_ref, n_pos, row_l):
      def body(p, accs):
        tok = plsc.load_gather(tok_ref, [row_l + p])
        t4 = tok * EP
        accs = list(accs)
        for jj in range(EP):
          g = plsc.load_gather(table_ref, [t4 + jj])
          lo, hi = _unpack2(g)
          accs[2 * jj] = accs[2 * jj] + lo
          accs[2 * jj + 1] = accs[2 * jj + 1] + hi
        return tuple(accs)
      return plsc.parallel_loop(0, n_pos, 1, unroll=4,
                                carry=(zero,) * E)(body)

    for b in range(2):
      pltpu.async_copy(pn_slice(b), pn_v.at[b], sem_pn[b])
      pltpu.async_copy(tnc_slice(b), tnc_v.at[b], sem_tnc[b])

    def outer(o, _):
      for b in range(2):
        s = 2 * o + b
        pltpu.make_async_copy(pn_slice(s), pn_v.at[b], sem_pn[b]).wait()
        pltpu.make_async_copy(tnc_slice(s), tnc_v.at[b], sem_tnc[b]).wait()

        @pl.when(o >= 1)
        def _wait_out():
          pltpu.make_async_copy(out_v.at[b], out_slice(s), sem_out[b]).wait()

        def group_body(g, _):
          rb = g * LANES
          row_pn = row_iota * L_PN + rb * L_PN
          row_tnc = row_iota * L_TNC + rb * L_TNC
          acc_pn = pool(pn_v.at[b], pt_v, L_PN, row_pn)
          acc_tnc = pool(tnc_v.at[b], tt_v, L_TNC, row_tnc)
          feats = acc_pn + acc_tnc

          orow = row_iota * D_OUT + rb * D_OUT

          def dense_body(j, _):
            parts = [bb_v[pl.ds(j * LANES, LANES)], None, None, None]
            for m in range(E):
              wlo, whi = _unpack2(
                  wb_v[pl.ds((m * D_OUT + j) * LANES, LANES)])
              t = feats[2 * m] * wlo + feats[2 * m + 1] * whi
              parts[m % 4] = t if parts[m % 4] is None else parts[m % 4] + t
            oj = (parts[0] + parts[1]) + (parts[2] + parts[3])
            oj = jnp.maximum(oj, 0.0)
            plsc.store_scatter(out_v.at[b], [orow + j], oj)
            return 0

          lax.fori_loop(0, D_OUT, dense_body, 0, unroll=4)
          return 0

        lax.fori_loop(0, NGRP, group_body, 0)

        pltpu.async_copy(out_v.at[b], out_slice(s), sem_out[b])

        @pl.when(s + 2 < NSG)
        def _prefetch():
          pltpu.async_copy(pn_slice(s + 2), pn_v.at[b], sem_pn[b])
          pltpu.async_copy(tnc_slice(s + 2), tnc_v.at[b], sem_tnc[b])
      return 0

    lax.fori_loop(0, NSG // 2, outer, 0)

    for b in range(2):
      pltpu.make_async_copy(out_v.at[b], out_slice(b), sem_out[b]).wait()

  return k


_sc_kernel = _make_kernel()


def _pack_bf16_pairs(x):
  xb = x.astype(jnp.bfloat16).reshape(x.shape[0], -1, 2)
  return lax.bitcast_convert_type(xb, jnp.int32).reshape(-1)


@jax.jit
def kernel(program_name_tokens, tnc_tokens, program_table, tnc_table, W, b):
  scale = jnp.concatenate([
      jnp.full((E,), 1.0 / L_PN, jnp.float32),
      jnp.full((E,), 1.0 / L_TNC, jnp.float32),
  ])
  Ws = W * scale[:, None]
  wpairs = Ws.reshape(E, 2, D_OUT).transpose(0, 2, 1)
  wp = _pack_bf16_pairs(wpairs.reshape(E * D_OUT, 2))
  wb = jnp.tile(wp[:, None], (1, LANES)).reshape(-1)
  bb = jnp.tile(b[:, None], (1, LANES)).reshape(-1)
  out = _sc_kernel(
      program_name_tokens.astype(jnp.int32).reshape(-1),
      tnc_tokens.astype(jnp.int32).reshape(-1),
      _pack_bf16_pairs(program_table),
      _pack_bf16_pairs(tnc_table),
      wb,
      bb,
  )
  return out.reshape(B, D_OUT)
